# Initial kernel scaffold; baseline (speedup 1.0000x reference)
import functools

import jax
import jax.numpy as jnp
from jax import lax
from jax.experimental import pallas as pl
from jax.experimental.pallas import tpu as pltpu
from jax.experimental.pallas import tpu_sc as plsc

NUM_EMB = 1_000_000
D = 32
B = 16384 * 50
NW = 32
B_PER_W = B // NW
CH = 1280
NCH = B_PER_W // CH

_mesh = plsc.VectorSubcoreMesh(core_axis_name="c", subcore_axis_name="s")


@functools.partial(
    pl.kernel,
    mesh=_mesh,
    out_type=jax.ShapeDtypeStruct((B, D), jnp.float32),
    scratch_types=[
        pltpu.VMEM((CH,), jnp.int32),
        pltpu.VMEM((CH, D), jnp.float32),
        pltpu.SemaphoreType.DMA,
    ],
)
def _gather(idx_hbm, table_hbm, out_hbm, idx_v, rows_v, sem):
    wid = lax.axis_index("s") * 2 + lax.axis_index("c")
    base = wid * B_PER_W

    def body(j, carry):
        off = base + j * CH
        pltpu.sync_copy(idx_hbm.at[pl.ds(off, CH)], idx_v)
        pltpu.async_copy(table_hbm.at[idx_v], rows_v, sem).wait()
        pltpu.sync_copy(rows_v, out_hbm.at[pl.ds(off, CH)])
        return carry

    lax.fori_loop(0, NCH, body, 0)


def kernel(input, table):
    flat = input.reshape(-1).astype(jnp.int32)
    out = _gather(flat, table)
    return out.reshape(input.shape + (D,))


# speedup vs baseline: 1.1002x; 1.1002x over previous
import functools

import jax
import jax.numpy as jnp
from jax import lax
from jax.experimental import pallas as pl
from jax.experimental.pallas import tpu as pltpu
from jax.experimental.pallas import tpu_sc as plsc

NUM_EMB = 1_000_000
D = 32
B = 16384 * 50
NW = 32
B_PER_W = B // NW
CH = 1280
NCH = B_PER_W // CH

_mesh = plsc.VectorSubcoreMesh(core_axis_name="c", subcore_axis_name="s")


@functools.partial(
    pl.kernel,
    mesh=_mesh,
    out_type=jax.ShapeDtypeStruct((B, D), jnp.float32),
    scratch_types=[
        pltpu.VMEM((CH,), jnp.int32),
        pltpu.VMEM((CH, D), jnp.float32),
        pltpu.SemaphoreType.DMA,
    ],
    compiler_params=pltpu.CompilerParams(use_tc_tiling_on_sc=False),
)
def _gather(idx_hbm, table_hbm, out_hbm, idx_v, rows_v, sem):
    wid = lax.axis_index("s") * 2 + lax.axis_index("c")
    base = wid * B_PER_W

    def body(j, carry):
        off = base + j * CH
        pltpu.sync_copy(idx_hbm.at[pl.ds(off, CH)], idx_v)
        pltpu.async_copy(table_hbm.at[idx_v], rows_v, sem).wait()
        pltpu.sync_copy(rows_v, out_hbm.at[pl.ds(off, CH)])
        return carry

    lax.fori_loop(0, NCH, body, 0)


def kernel(input, table):
    flat = input.reshape(-1).astype(jnp.int32)
    out = _gather(flat, table)
    return out.reshape(input.shape + (D,))


# speedup vs baseline: 1.1129x; 1.0116x over previous
import functools

import jax
import jax.numpy as jnp
from jax import lax
from jax.experimental import pallas as pl
from jax.experimental.pallas import tpu as pltpu
from jax.experimental.pallas import tpu_sc as plsc

NUM_EMB = 1_000_000
D = 32
B = 16384 * 50
NW = 32
B_PER_W = B // NW
CH = 640
NCH = B_PER_W // CH
NB = 4
DP = 2

_mesh = plsc.VectorSubcoreMesh(core_axis_name="c", subcore_axis_name="s")


@functools.partial(
    pl.kernel,
    mesh=_mesh,
    out_type=jax.ShapeDtypeStruct((B, D), jnp.float32),
    scratch_types=[
        pltpu.VMEM((B_PER_W,), jnp.int32),
        [pltpu.VMEM((CH, D), jnp.float32) for _ in range(NB)],
        [pltpu.SemaphoreType.DMA for _ in range(NB)],
        [pltpu.SemaphoreType.DMA for _ in range(NB)],
    ],
    compiler_params=pltpu.CompilerParams(use_tc_tiling_on_sc=False),
)
def _gather(idx_hbm, table_hbm, out_hbm, idx_all, rows, gsem, ssem):
    wid = lax.axis_index("s") * 2 + lax.axis_index("c")
    base = wid * B_PER_W

    pltpu.sync_copy(idx_hbm.at[pl.ds(base, B_PER_W)], idx_all)

    def gather_descr(g, slot):
        src = table_hbm.at[idx_all.at[pl.ds(g * CH, CH)]]
        return pltpu.make_async_copy(src, rows[slot], gsem[slot])

    def store_descr(g, slot):
        dst = out_hbm.at[pl.ds(base + g * CH, CH)]
        return pltpu.make_async_copy(rows[slot], dst, ssem[slot])

    for b in range(DP):
        gather_descr(b, b).start()

    @pl.loop(0, NCH, step=NB)
    def _(j):
        for b in range(NB):
            g = j + b
            gp = g + DP
            pb = (b + DP) % NB

            @pl.when(gp < NCH)
            def _():
                @pl.when(gp >= NB)
                def _():
                    store_descr(gp - NB, pb).wait()

                gather_descr(gp, pb).start()

            gather_descr(g, b).wait()
            store_descr(g, b).start()

    for b in range(NB):
        store_descr(NCH - NB + b, b).wait()


def kernel(input, table):
    flat = input.reshape(-1).astype(jnp.int32)
    out = _gather(flat, table)
    return out.reshape(input.shape + (D,))


# speedup vs baseline: 1.8056x; 1.6224x over previous
import functools

import jax
import jax.numpy as jnp
from jax import lax
from jax.experimental import pallas as pl
from jax.experimental.pallas import tpu as pltpu
from jax.experimental.pallas import tpu_sc as plsc

NUM_EMB = 1_000_000
D = 32
NROW = 16384
HIST = 50
NW = 32
ROWS_PER_W = NROW // NW
GR = 8
NCH = ROWS_PER_W // GR
NB = 4
DP = 2

_mesh = plsc.VectorSubcoreMesh(core_axis_name="c", subcore_axis_name="s")


@functools.partial(
    pl.kernel,
    mesh=_mesh,
    out_type=jax.ShapeDtypeStruct((NROW, HIST, D), jnp.float32),
    scratch_types=[
        pltpu.VMEM((ROWS_PER_W, HIST), jnp.int32),
        [pltpu.VMEM((GR, HIST, D), jnp.float32) for _ in range(NB)],
        [pltpu.SemaphoreType.DMA for _ in range(NB)],
        [pltpu.SemaphoreType.DMA for _ in range(NB)],
    ],
    compiler_params=pltpu.CompilerParams(use_tc_tiling_on_sc=False),
)
def _gather(idx_hbm, table_hbm, out_hbm, idx_all, rows, gsem, ssem):
    wid = lax.axis_index("s") * 2 + lax.axis_index("c")
    base = wid * ROWS_PER_W

    pltpu.sync_copy(idx_hbm.at[pl.ds(base, ROWS_PER_W)], idx_all)

    def row_gather_descr(c, r, slot):
        src = table_hbm.at[idx_all.at[c * GR + r]]
        return pltpu.make_async_copy(src, rows[slot].at[r], gsem[slot])

    def store_descr(c, slot):
        dst = out_hbm.at[pl.ds(base + c * GR, GR)]
        return pltpu.make_async_copy(rows[slot], dst, ssem[slot])

    for b in range(DP):
        for r in range(GR):
            row_gather_descr(b, r, b).start()

    @pl.loop(0, NCH, step=NB)
    def _(j):
        for b in range(NB):
            c = j + b
            cp = c + DP
            pb = (b + DP) % NB

            @pl.when(cp < NCH)
            def _():
                @pl.when(cp >= NB)
                def _():
                    store_descr(cp - NB, pb).wait()

                for r in range(GR):
                    row_gather_descr(cp, r, pb).start()

            for r in range(GR):
                row_gather_descr(c, r, b).wait()
            store_descr(c, b).start()

    for b in range(NB):
        store_descr(NCH - NB + b, b).wait()


def kernel(input, table):
    return _gather(input.astype(jnp.int32), table)
